# baseline (speedup 1.0000x reference)
import functools

import jax
import jax.numpy as jnp
from jax import lax
from jax.experimental import pallas as pl
from jax.experimental.pallas import tpu as pltpu
from jax.experimental.pallas import tpu_sc as plsc

VOCAB = 1000000
EMBED_DIM = 64
SEQ = 200
BATCH = 4096
NUM_WORKERS = 32

_MESH = dict(core_axis_name="c", subcore_axis_name="s")
_D_CHUNK = 32


_WV = 128
_N_FULL = VOCAB // _WV
_TAIL = VOCAB - _N_FULL * _WV
_TAIL_WORKER = 5
_UPW = (_N_FULL + NUM_WORKERS - 1) // NUM_WORKERS


@functools.partial(
    pl.kernel,
    mesh=plsc.VectorSubcoreMesh(**_MESH),
    compiler_params=pltpu.CompilerParams(needs_layout_passes=False),
    out_type=jax.ShapeDtypeStruct((VOCAB // 2, 2 * EMBED_DIM), jnp.float32),
    scratch_types=[
        pltpu.VMEM((EMBED_DIM, _WV), jnp.float32),
        pltpu.VMEM((EMBED_DIM, _WV), jnp.float32),
        pltpu.VMEM((_WV // 2, 2 * EMBED_DIM), jnp.float32),
        pltpu.VMEM((_WV // 2, 2 * EMBED_DIM), jnp.float32),
        pltpu.SemaphoreType.DMA,
        pltpu.SemaphoreType.DMA,
    ],
)
def _relayout(emb_t, tail_blk, table_rm, eb0, eb1, ov0, ov1, in_sem, out_sem):
    wid = lax.axis_index("s") * 2 + lax.axis_index("c")
    iota = lax.iota(jnp.int32, 16)
    rows = [lax.shift_right_logical(iota + 16 * m, 1) for m in range(8)]
    colb = [((iota + 16 * m) & 1) * EMBED_DIM for m in range(8)]

    def n_of(u):
        return jnp.minimum(wid * _UPW + u, _N_FULL - 1)

    def start_in(u, eb):
        pltpu.async_copy(emb_t.at[:, pl.ds(n_of(u) * _WV, _WV)], eb, in_sem)

    def wait_in(eb):
        pltpu.make_async_copy(emb_t.at[:, pl.ds(0, _WV)], eb, in_sem).wait()

    def start_out(u, ov):
        pltpu.async_copy(ov, table_rm.at[pl.ds(n_of(u) * (_WV // 2), _WV // 2)],
                         out_sem)

    def wait_out(ov):
        pltpu.make_async_copy(
            ov, table_rm.at[pl.ds(0, _WV // 2)], out_sem).wait()

    def transpose(eb, ov):
        @plsc.parallel_loop(0, EMBED_DIM, unroll=_D_CHUNK)
        def _body(dd):
            for m in range(8):
                vec = eb[dd, pl.ds(16 * m, 16)]
                plsc.store_scatter(ov, [rows[m], colb[m] + dd], vec)

    def unit(u, g, eb, ov, eb_next):
        start_in(u + 1, eb_next)
        wait_in(eb)

        @pl.when(g >= 1)
        def _():
            wait_out(ov)

        transpose(eb, ov)
        start_out(u, ov)

    start_in(0, eb0)

    def pair_body(g, carry):
        unit(2 * g, g, eb0, ov0, eb1)
        unit(2 * g + 1, g, eb1, ov1, eb0)
        return carry

    lax.fori_loop(0, _UPW // 2, pair_body, 0)
    u_last = _UPW - 1
    wait_in(eb0)
    wait_out(ov0)
    transpose(eb0, ov0)
    start_out(u_last, ov0)
    wait_out(ov0)
    wait_out(ov1)

    @pl.when(wid == _TAIL_WORKER)
    def _tail():
        pltpu.sync_copy(tail_blk, ov0.at[pl.ds(0, _TAIL // 2)])
        pltpu.sync_copy(ov0.at[pl.ds(0, _TAIL // 2)],
                        table_rm.at[pl.ds(_N_FULL * (_WV // 2), _TAIL // 2)])


_BW = 128


@functools.partial(
    pl.kernel,
    mesh=plsc.VectorSubcoreMesh(**_MESH),
    compiler_params=pltpu.CompilerParams(needs_layout_passes=False),
    out_type=jax.ShapeDtypeStruct((SEQ, EMBED_DIM, BATCH), jnp.float32),
    scratch_types=[
        pltpu.VMEM((SEQ, _BW), jnp.int32),
        pltpu.VMEM((_BW,), jnp.int32),
        pltpu.VMEM((_BW,), jnp.int32),
        pltpu.VMEM((_BW,), jnp.int32),
        pltpu.VMEM((_BW,), jnp.int32),
        pltpu.VMEM((_BW, 2 * EMBED_DIM), jnp.float32),
        pltpu.VMEM((_BW, 2 * EMBED_DIM), jnp.float32),
        pltpu.VMEM((EMBED_DIM, _BW), jnp.float32),
        pltpu.VMEM((EMBED_DIM, _BW), jnp.float32),
        pltpu.SemaphoreType.DMA,
        pltpu.SemaphoreType.DMA,
    ],
)
def _gather(idx_t, table_rm, out, idxw_v, ip0, ip1, cb0, cb1, r0, r1, ob0, ob1,
            in_sem, out_sem):
    wid = lax.axis_index("s") * 2 + lax.axis_index("c")
    b0 = wid * _BW
    pltpu.sync_copy(idx_t.at[:, pl.ds(b0, _BW)], idxw_v)
    iota = lax.iota(jnp.int32, 16)
    rows = [iota + 16 * m for m in range(8)]

    def prep(u, ip, cb):
        t = jnp.minimum(u, SEQ - 1)
        for k in range(_BW // 16):
            v = idxw_v[t, pl.ds(16 * k, 16)]
            ip[pl.ds(16 * k, 16)] = lax.shift_right_logical(v, 1)
            cb[pl.ds(16 * k, 16)] = (v & 1) * EMBED_DIM

    def start_gather(ip, rv):
        pltpu.async_copy(table_rm.at[ip], rv, in_sem)

    def wait_gather(ip, rv):
        pltpu.make_async_copy(table_rm.at[ip], rv, in_sem).wait()

    def start_out(u, ob):
        pltpu.async_copy(ob, out.at[u, :, pl.ds(b0, _BW)], out_sem)

    def wait_out(ob):
        pltpu.make_async_copy(ob, out.at[0, :, pl.ds(b0, _BW)], out_sem).wait()

    def transpose(rv, cb, ob):
        colb = [cb[pl.ds(16 * m, 16)] for m in range(8)]

        @plsc.parallel_loop(0, EMBED_DIM, unroll=_D_CHUNK)
        def _body(dd):
            for m in range(8):
                vec = plsc.load_gather(rv, [rows[m], colb[m] + dd])
                ob[dd, pl.ds(16 * m, 16)] = vec

    def unit(u, g, ip, cb, rv, ob, ip_n, cb_n, rv_n):
        prep(u + 1, ip_n, cb_n)
        start_gather(ip_n, rv_n)
        wait_gather(ip, rv)

        @pl.when(g >= 1)
        def _():
            wait_out(ob)

        transpose(rv, cb, ob)
        start_out(u, ob)

    prep(0, ip0, cb0)
    start_gather(ip0, r0)

    def pair_body(g, carry):
        unit(2 * g, g, ip0, cb0, r0, ob0, ip1, cb1, r1)
        unit(2 * g + 1, g, ip1, cb1, r1, ob1, ip0, cb0, r0)
        return carry

    lax.fori_loop(0, SEQ // 2, pair_body, 0)
    wait_gather(ip0, r0)
    wait_out(ob0)
    wait_out(ob1)


def kernel(inputs, embeddings):
    emb_t = embeddings.T
    idx_t = inputs.T
    tail_blk = jnp.reshape(
        lax.slice(embeddings, (VOCAB - _TAIL, 0), (VOCAB, EMBED_DIM)),
        (_TAIL // 2, 2 * EMBED_DIM))
    table_rm = _relayout(emb_t, tail_blk)
    out_t = _gather(idx_t, table_rm)
    return jnp.transpose(out_t, (2, 0, 1))


# speedup vs baseline: 1.0317x; 1.0317x over previous
import functools

import jax
import jax.numpy as jnp
from jax import lax
from jax.experimental import pallas as pl
from jax.experimental.pallas import tpu as pltpu
from jax.experimental.pallas import tpu_sc as plsc

VOCAB = 1000000
EMBED_DIM = 64
SEQ = 200
BATCH = 4096
NUM_WORKERS = 32

_MESH = dict(core_axis_name="c", subcore_axis_name="s")
_D_CHUNK = 16


_WV = 128
_N_FULL = VOCAB // _WV
_TAIL = VOCAB - _N_FULL * _WV
_TAIL_WORKER = 5
_UPW = (_N_FULL + NUM_WORKERS - 1) // NUM_WORKERS


@functools.partial(
    pl.kernel,
    mesh=plsc.VectorSubcoreMesh(**_MESH),
    compiler_params=pltpu.CompilerParams(needs_layout_passes=False),
    out_type=jax.ShapeDtypeStruct((VOCAB // 2, 2 * EMBED_DIM), jnp.float32),
    scratch_types=[
        pltpu.VMEM((EMBED_DIM, _WV), jnp.float32),
        pltpu.VMEM((EMBED_DIM, _WV), jnp.float32),
        pltpu.VMEM((_WV // 2, 2 * EMBED_DIM), jnp.float32),
        pltpu.VMEM((_WV // 2, 2 * EMBED_DIM), jnp.float32),
        pltpu.SemaphoreType.DMA,
        pltpu.SemaphoreType.DMA,
    ],
)
def _relayout(emb_t, tail_blk, table_rm, eb0, eb1, ov0, ov1, in_sem, out_sem):
    wid = lax.axis_index("s") * 2 + lax.axis_index("c")
    iota = lax.iota(jnp.int32, 16)
    rows = [lax.shift_right_logical(iota + 16 * m, 1) for m in range(8)]
    colb = [((iota + 16 * m) & 1) * EMBED_DIM for m in range(8)]

    def n_of(u):
        return jnp.minimum(wid * _UPW + u, _N_FULL - 1)

    def start_in(u, eb):
        pltpu.async_copy(emb_t.at[:, pl.ds(n_of(u) * _WV, _WV)], eb, in_sem)

    def wait_in(eb):
        pltpu.make_async_copy(emb_t.at[:, pl.ds(0, _WV)], eb, in_sem).wait()

    def start_out(u, ov):
        pltpu.async_copy(ov, table_rm.at[pl.ds(n_of(u) * (_WV // 2), _WV // 2)],
                         out_sem)

    def wait_out(ov):
        pltpu.make_async_copy(
            ov, table_rm.at[pl.ds(0, _WV // 2)], out_sem).wait()

    def transpose(eb, ov):
        @plsc.parallel_loop(0, EMBED_DIM, unroll=_D_CHUNK)
        def _body(dd):
            for m in range(8):
                vec = eb[dd, pl.ds(16 * m, 16)]
                plsc.store_scatter(ov, [rows[m], colb[m] + dd], vec)

    def unit(u, g, eb, ov, eb_next):
        start_in(u + 1, eb_next)
        wait_in(eb)

        @pl.when(g >= 1)
        def _():
            wait_out(ov)

        transpose(eb, ov)
        start_out(u, ov)

    start_in(0, eb0)

    def pair_body(g, carry):
        unit(2 * g, g, eb0, ov0, eb1)
        unit(2 * g + 1, g, eb1, ov1, eb0)
        return carry

    lax.fori_loop(0, _UPW // 2, pair_body, 0)
    u_last = _UPW - 1
    wait_in(eb0)
    wait_out(ov0)
    transpose(eb0, ov0)
    start_out(u_last, ov0)
    wait_out(ov0)
    wait_out(ov1)

    @pl.when(wid == _TAIL_WORKER)
    def _tail():
        pltpu.sync_copy(tail_blk, ov0.at[pl.ds(0, _TAIL // 2)])
        pltpu.sync_copy(ov0.at[pl.ds(0, _TAIL // 2)],
                        table_rm.at[pl.ds(_N_FULL * (_WV // 2), _TAIL // 2)])


_BW = 128


@functools.partial(
    pl.kernel,
    mesh=plsc.VectorSubcoreMesh(**_MESH),
    compiler_params=pltpu.CompilerParams(needs_layout_passes=False),
    out_type=jax.ShapeDtypeStruct((SEQ, EMBED_DIM, BATCH), jnp.float32),
    scratch_types=[
        pltpu.VMEM((SEQ, _BW), jnp.int32),
        pltpu.VMEM((_BW,), jnp.int32),
        pltpu.VMEM((_BW,), jnp.int32),
        pltpu.VMEM((_BW,), jnp.int32),
        pltpu.VMEM((_BW,), jnp.int32),
        pltpu.VMEM((_BW, 2 * EMBED_DIM), jnp.float32),
        pltpu.VMEM((_BW, 2 * EMBED_DIM), jnp.float32),
        pltpu.VMEM((EMBED_DIM, _BW), jnp.float32),
        pltpu.VMEM((EMBED_DIM, _BW), jnp.float32),
        pltpu.SemaphoreType.DMA,
        pltpu.SemaphoreType.DMA,
    ],
)
def _gather(idx_t, table_rm, out, idxw_v, ip0, ip1, cb0, cb1, r0, r1, ob0, ob1,
            in_sem, out_sem):
    wid = lax.axis_index("s") * 2 + lax.axis_index("c")
    b0 = wid * _BW
    pltpu.sync_copy(idx_t.at[:, pl.ds(b0, _BW)], idxw_v)
    iota = lax.iota(jnp.int32, 16)
    rows = [iota + 16 * m for m in range(8)]

    def prep(u, ip, cb):
        t = jnp.minimum(u, SEQ - 1)
        for k in range(_BW // 16):
            v = idxw_v[t, pl.ds(16 * k, 16)]
            ip[pl.ds(16 * k, 16)] = lax.shift_right_logical(v, 1)
            cb[pl.ds(16 * k, 16)] = (v & 1) * EMBED_DIM

    def start_gather(ip, rv):
        pltpu.async_copy(table_rm.at[ip], rv, in_sem)

    def wait_gather(ip, rv):
        pltpu.make_async_copy(table_rm.at[ip], rv, in_sem).wait()

    def start_out(u, ob):
        pltpu.async_copy(ob, out.at[u, :, pl.ds(b0, _BW)], out_sem)

    def wait_out(ob):
        pltpu.make_async_copy(ob, out.at[0, :, pl.ds(b0, _BW)], out_sem).wait()

    def transpose(rv, cb, ob):
        colb = [cb[pl.ds(16 * m, 16)] for m in range(8)]

        @plsc.parallel_loop(0, EMBED_DIM, unroll=_D_CHUNK)
        def _body(dd):
            for m in range(8):
                vec = plsc.load_gather(rv, [rows[m], colb[m] + dd])
                ob[dd, pl.ds(16 * m, 16)] = vec

    def unit(u, g, ip, cb, rv, ob, ip_n, cb_n, rv_n):
        prep(u + 1, ip_n, cb_n)
        start_gather(ip_n, rv_n)
        wait_gather(ip, rv)

        @pl.when(g >= 1)
        def _():
            wait_out(ob)

        transpose(rv, cb, ob)
        start_out(u, ob)

    prep(0, ip0, cb0)
    start_gather(ip0, r0)

    def pair_body(g, carry):
        unit(2 * g, g, ip0, cb0, r0, ob0, ip1, cb1, r1)
        unit(2 * g + 1, g, ip1, cb1, r1, ob1, ip0, cb0, r0)
        return carry

    lax.fori_loop(0, SEQ // 2, pair_body, 0)
    wait_gather(ip0, r0)
    wait_out(ob0)
    wait_out(ob1)


def kernel(inputs, embeddings):
    emb_t = embeddings.T
    idx_t = inputs.T
    tail_blk = jnp.reshape(
        lax.slice(embeddings, (VOCAB - _TAIL, 0), (VOCAB, EMBED_DIM)),
        (_TAIL // 2, 2 * EMBED_DIM))
    table_rm = _relayout(emb_t, tail_blk)
    out_t = _gather(idx_t, table_rm)
    return jnp.transpose(out_t, (2, 0, 1))
